# baseline (speedup 1.0000x reference)
import functools

import jax
import jax.numpy as jnp
from jax import lax
from jax.experimental import pallas as pl
from jax.experimental.pallas import tpu as pltpu
from jax.experimental.pallas import tpu_sc as plsc

NUM_DOSES = 100
EMB_DIM = 64
BATCH = 16384

NC, NS, L = 2, 16, 16
NW = NC * NS
B_PER_W = BATCH // NW

MLP_BLK = 2048


def _sc_gather_build():
    mesh = plsc.VectorSubcoreMesh(core_axis_name="c", subcore_axis_name="s")

    @functools.partial(
        pl.kernel,
        mesh=mesh,
        out_type=jax.ShapeDtypeStruct((BATCH, EMB_DIM), jnp.float32),
        scratch_types=[
            pltpu.VMEM((B_PER_W,), jnp.int32),
            pltpu.VMEM((B_PER_W,), jnp.int32),
            pltpu.VMEM((B_PER_W, EMB_DIM), jnp.float32),
            pltpu.SemaphoreType.DMA,
        ],
        compiler_params=pltpu.CompilerParams(use_tc_tiling_on_sc=False),
    )
    def sc_gather(dev_hbm, dose_hbm, table_hbm, out_hbm, dev_v, idx_v, rows_v, sem):
        wid = lax.axis_index("s") * NC + lax.axis_index("c")
        base = wid * B_PER_W
        pltpu.sync_copy(dev_hbm.at[pl.ds(base, B_PER_W)], dev_v)
        pltpu.sync_copy(dose_hbm.at[pl.ds(base, B_PER_W)], idx_v)

        @pl.loop(0, B_PER_W, step=L)
        def _(i):
            s = pl.ds(i, L)
            idx_v[s] = dev_v[s] * NUM_DOSES + idx_v[s]

        pltpu.async_copy(table_hbm.at[idx_v], rows_v, sem).wait()
        pltpu.sync_copy(rows_v, out_hbm.at[pl.ds(base, B_PER_W)])

    return sc_gather


_sc_gather = _sc_gather_build()


def _mlp_body(emb_ref, w1_ref, b1_ref, w2_ref, b2_ref, out_ref):
    emb = emb_ref[...]
    h = jnp.dot(emb, w1_ref[...], preferred_element_type=jnp.float32)
    h = h + b1_ref[...]
    h = 0.5 * h * (1.0 + lax.erf(h * 0.7071067811865476))
    out = jnp.dot(h, w2_ref[...], preferred_element_type=jnp.float32)
    out_ref[...] = out + b2_ref[...] + emb


def _tc_mlp(emb, W1, b1, W2, b2):
    return pl.pallas_call(
        _mlp_body,
        grid=(BATCH // MLP_BLK,),
        in_specs=[
            pl.BlockSpec((MLP_BLK, EMB_DIM), lambda i: (i, 0)),
            pl.BlockSpec((EMB_DIM, 2 * EMB_DIM), lambda i: (0, 0)),
            pl.BlockSpec((1, 2 * EMB_DIM), lambda i: (0, 0)),
            pl.BlockSpec((2 * EMB_DIM, EMB_DIM), lambda i: (0, 0)),
            pl.BlockSpec((1, EMB_DIM), lambda i: (0, 0)),
        ],
        out_specs=pl.BlockSpec((MLP_BLK, EMB_DIM), lambda i: (i, 0)),
        out_shape=jax.ShapeDtypeStruct((BATCH, EMB_DIM), jnp.float32),
    )(emb, W1, b1, W2, b2)


@jax.jit
def kernel(table, W1, b1, W2, b2, device_idx, dose_idx):
    dev = device_idx.astype(jnp.int32)
    dose = dose_idx.astype(jnp.int32)
    emb = _sc_gather(dev, dose, table)
    return _tc_mlp(emb, W1, b1.reshape(1, -1), W2, b2.reshape(1, -1))


# speedup vs baseline: 1.6876x; 1.6876x over previous
import functools

import jax
import jax.numpy as jnp
from jax import lax
from jax.experimental import pallas as pl
from jax.experimental.pallas import tpu as pltpu
from jax.experimental.pallas import tpu_sc as plsc

NUM_DOSES = 100
EMB_DIM = 64
BATCH = 16384

NC, NS, L = 2, 16, 16
NW = NC * NS
B_PER_W = BATCH // NW

MLP_BLK = 2048


def _sc_gather_build():
    mesh = plsc.VectorSubcoreMesh(core_axis_name="c", subcore_axis_name="s")

    @functools.partial(
        pl.kernel,
        mesh=mesh,
        out_type=jax.ShapeDtypeStruct((BATCH, EMB_DIM), jnp.float32),
        scratch_types=[
            pltpu.VMEM((B_PER_W,), jnp.int32),
            pltpu.VMEM((B_PER_W,), jnp.int32),
            pltpu.VMEM((B_PER_W, EMB_DIM), jnp.float32),
            pltpu.SemaphoreType.DMA,
        ],
        compiler_params=pltpu.CompilerParams(use_tc_tiling_on_sc=True),
    )
    def sc_gather(dev_hbm, dose_hbm, table_hbm, out_hbm, dev_v, idx_v, rows_v, sem):
        wid = lax.axis_index("s") * NC + lax.axis_index("c")
        base = wid * B_PER_W
        pltpu.sync_copy(dev_hbm.at[pl.ds(base, B_PER_W)], dev_v)
        pltpu.sync_copy(dose_hbm.at[pl.ds(base, B_PER_W)], idx_v)

        @pl.loop(0, B_PER_W, step=L)
        def _(i):
            s = pl.ds(i, L)
            idx_v[s] = dev_v[s] * NUM_DOSES + idx_v[s]

        @pl.loop(0, B_PER_W, step=L)
        def _(g):
            v = idx_v[pl.ds(g, L)]
            for k in range(L):
                pltpu.make_async_copy(
                    table_hbm.at[pl.ds(v[k], 1)],
                    rows_v.at[pl.ds(g + k, 1)],
                    sem,
                ).start()

        @pl.loop(0, B_PER_W)
        def _(j):
            pltpu.make_async_copy(
                table_hbm.at[pl.ds(0, 1)], rows_v.at[pl.ds(j, 1)], sem
            ).wait()

        pltpu.sync_copy(rows_v, out_hbm.at[pl.ds(base, B_PER_W)])

    return sc_gather


_sc_gather = _sc_gather_build()


def _mlp_body(emb_ref, w1_ref, b1_ref, w2_ref, b2_ref, out_ref):
    emb = emb_ref[...]
    h = jnp.dot(emb, w1_ref[...], preferred_element_type=jnp.float32)
    h = h + b1_ref[...]
    h = 0.5 * h * (1.0 + lax.erf(h * 0.7071067811865476))
    out = jnp.dot(h, w2_ref[...], preferred_element_type=jnp.float32)
    out_ref[...] = out + b2_ref[...] + emb


def _tc_mlp(emb, W1, b1, W2, b2):
    return pl.pallas_call(
        _mlp_body,
        grid=(BATCH // MLP_BLK,),
        in_specs=[
            pl.BlockSpec((MLP_BLK, EMB_DIM), lambda i: (i, 0)),
            pl.BlockSpec((EMB_DIM, 2 * EMB_DIM), lambda i: (0, 0)),
            pl.BlockSpec((1, 2 * EMB_DIM), lambda i: (0, 0)),
            pl.BlockSpec((2 * EMB_DIM, EMB_DIM), lambda i: (0, 0)),
            pl.BlockSpec((1, EMB_DIM), lambda i: (0, 0)),
        ],
        out_specs=pl.BlockSpec((MLP_BLK, EMB_DIM), lambda i: (i, 0)),
        out_shape=jax.ShapeDtypeStruct((BATCH, EMB_DIM), jnp.float32),
    )(emb, W1, b1, W2, b2)


@jax.jit
def kernel(table, W1, b1, W2, b2, device_idx, dose_idx):
    dev = device_idx.astype(jnp.int32)
    dose = dose_idx.astype(jnp.int32)
    emb = _sc_gather(dev, dose, table)
    return _tc_mlp(emb, W1, b1.reshape(1, -1), W2, b2.reshape(1, -1))
